# Initial kernel scaffold; baseline (speedup 1.0000x reference)
import jax
import jax.numpy as jnp
from jax.experimental import pallas as pl


def kernel(input_ids, word_emb, pos_emb, type_emb, ln_gamma, ln_beta):
    raise NotImplementedError("write your pallas kernel here")


# speedup vs baseline: 2.1327x; 2.1327x over previous
import functools

import jax
import jax.numpy as jnp
from jax import lax
from jax.experimental import pallas as pl
from jax.experimental.pallas import tpu as pltpu
from jax.experimental.pallas import tpu_sc as plsc

B, S, H = 128, 512, 768
EPS = 1e-12

NUM_WORKERS = 32
CHUNK = 64
TOK_PER_W = (B * S) // NUM_WORKERS
CHUNKS_PER_W = TOK_PER_W // CHUNK


def _sc_gather_body(table_hbm, idx_hbm, out_hbm, idx_v, rows_v,
                    gsem0, gsem1, ssem0, ssem1):
    gsems = (gsem0, gsem1)
    ssems = (ssem0, ssem1)
    wid = lax.axis_index("s") * 2 + lax.axis_index("c")
    row0 = wid * CHUNKS_PER_W
    pltpu.sync_copy(idx_hbm.at[pl.ds(row0, CHUNKS_PER_W)], idx_v)

    def start_gather(j):
        return pltpu.async_copy(
            table_hbm.at[idx_v.at[j]], rows_v.at[j % 2], gsems[j % 2])

    def start_store(j):
        return pltpu.async_copy(
            rows_v.at[j % 2],
            out_hbm.at[pl.ds((row0 + j) * CHUNK, CHUNK)],
            ssems[j % 2])

    gathers = [None] * CHUNKS_PER_W
    stores = [None] * CHUNKS_PER_W
    gathers[0] = start_gather(0)
    for j in range(CHUNKS_PER_W):
        nxt = j + 1
        if nxt < CHUNKS_PER_W:
            if nxt >= 2:
                stores[nxt - 2].wait()
            gathers[nxt] = start_gather(nxt)
        gathers[j].wait()
        stores[j] = start_store(j)
    stores[CHUNKS_PER_W - 2].wait()
    stores[CHUNKS_PER_W - 1].wait()


def _sc_gather(word_emb, ids2d):
    mesh = plsc.VectorSubcoreMesh(core_axis_name="c", subcore_axis_name="s")
    k = functools.partial(
        pl.kernel,
        mesh=mesh,
        out_type=jax.ShapeDtypeStruct((B * S, H), jnp.float32),
        scratch_types=[
            pltpu.VMEM((CHUNKS_PER_W, CHUNK), jnp.int32),
            pltpu.VMEM((2, CHUNK, H), jnp.float32),
            pltpu.SemaphoreType.DMA,
            pltpu.SemaphoreType.DMA,
            pltpu.SemaphoreType.DMA,
            pltpu.SemaphoreType.DMA,
        ],
    )(_sc_gather_body)
    return k(word_emb, ids2d)


def _ln_body(g_ref, pos_ref, type_ref, gamma_ref, beta_ref, o_ref):
    x = g_ref[0] + pos_ref[...] + type_ref[0]
    mean = jnp.mean(x, axis=-1, keepdims=True)
    cent = x - mean
    var = jnp.mean(cent * cent, axis=-1, keepdims=True)
    o_ref[0] = cent * lax.rsqrt(var + EPS) * gamma_ref[...] + beta_ref[...]


def _tc_layernorm(gathered, pos_emb, type_emb, ln_gamma, ln_beta):
    return pl.pallas_call(
        _ln_body,
        grid=(B,),
        in_specs=[
            pl.BlockSpec((1, S, H), lambda i: (i, 0, 0)),
            pl.BlockSpec((S, H), lambda i: (0, 0)),
            pl.BlockSpec((2, H), lambda i: (0, 0)),
            pl.BlockSpec((H,), lambda i: (0,)),
            pl.BlockSpec((H,), lambda i: (0,)),
        ],
        out_specs=pl.BlockSpec((1, S, H), lambda i: (i, 0, 0)),
        out_shape=jax.ShapeDtypeStruct((B, S, H), jnp.float32),
    )(gathered, pos_emb, type_emb, ln_gamma, ln_beta)


def kernel(input_ids, word_emb, pos_emb, type_emb, ln_gamma, ln_beta):
    ids2d = input_ids.reshape(-1, CHUNK)
    gathered = _sc_gather(word_emb, ids2d)
    return _tc_layernorm(
        gathered.reshape(B, S, H), pos_emb, type_emb, ln_gamma, ln_beta
    )
